# Initial kernel scaffold; baseline (speedup 1.0000x reference)
import functools

import jax
import jax.numpy as jnp
from jax import lax
from jax.experimental import pallas as pl
from jax.experimental.pallas import tpu as pltpu
from jax.experimental.pallas import tpu_sc as plsc

NCORES = 2
NSUB = 16
LANES = 16

CHUNK = 16384
BLK = 128
BLK_SHIFT = 7


@functools.partial(jax.jit, static_argnames=("p_per_tile",))
def _scatter_add(mem, val, idx2d, p_per_tile):
    s_rows, feat = mem.shape
    nchunks = s_rows // CHUNK
    ch_per_core = nchunks // NCORES
    p = p_per_tile
    nblk = (p + BLK - 1) // BLK
    rows_per_tile = CHUNK // NSUB
    trash = CHUNK

    mesh = plsc.VectorSubcoreMesh(
        core_axis_name="c", subcore_axis_name="s",
        num_cores=NCORES, num_subcores=NSUB)

    @pl.kernel(
        out_type=jax.ShapeDtypeStruct((s_rows, feat), jnp.float32),
        mesh=mesh,
        scratch_types=[
            pltpu.VMEM((p,), jnp.int32),
            pltpu.VMEM((nblk, BLK), jnp.int32),
            pltpu.VMEM((nblk, BLK), jnp.int32),
            pltpu.VMEM((BLK, feat), jnp.float32),
            pltpu.VMEM_SHARED((CHUNK + 8, feat), jnp.float32),
        ],
    )
    def scatter_kernel(mem_hbm, val_hbm, idx_hbm, out_hbm,
                       idx_cache, pid_list, loc_list, rows_buf, acc):
        core = lax.axis_index("c")
        sub = lax.axis_index("s")

        pltpu.sync_copy(idx_hbm.at[sub], idx_cache)

        ones = jnp.ones((LANES,), jnp.int32)
        zeros = jnp.zeros((LANES,), jnp.int32)
        trash_v = jnp.full((LANES,), trash, jnp.int32)
        lane = lax.iota(jnp.int32, LANES)

        @pl.loop(0, ch_per_core)
        def _chunk(c):
            base = (core * ch_per_core + c) * CHUNK

            pltpu.sync_copy(
                mem_hbm.at[pl.ds(base + sub * rows_per_tile, rows_per_tile)],
                acc.at[pl.ds(sub * rows_per_tile, rows_per_tile)])
            plsc.subcore_barrier()

            def scan_body(g, n):
                v = idx_cache[pl.ds(g * LANES, LANES)]
                loc = v - base
                mask = (loc >= 0) & (loc < CHUNK)
                mi = jnp.where(mask, ones, zeros)
                pos = n + plsc.cumsum(mi) - 1
                row = lax.shift_right_logical(pos, BLK_SHIFT)
                col = lax.bitwise_and(pos, BLK - 1)
                pid = sub * p + g * LANES + lane
                plsc.store_scatter(pid_list, [row, col], pid, mask)
                plsc.store_scatter(loc_list, [row, col], loc, mask)
                return n + jnp.sum(mi)

            n = lax.fori_loop(0, p // LANES, scan_body, jnp.int32(0),
                              unroll=4)

            nb = lax.shift_right_logical(n + (BLK - 1), BLK_SHIFT)
            lim = nb * BLK
            for j in range(BLK // LANES):
                pos = n + j * LANES + lane
                mask = pos < lim
                row = lax.shift_right_logical(pos, BLK_SHIFT)
                col = lax.bitwise_and(pos, BLK - 1)
                plsc.store_scatter(pid_list, [row, col], zeros, mask)
                plsc.store_scatter(loc_list, [row, col], trash_v, mask)

            @pl.loop(0, nblk)
            def _blk(b):
                @pl.when(b < nb)
                def _():
                    pltpu.sync_copy(val_hbm.at[pid_list.at[b]], rows_buf)
                    pltpu.sync_copy(rows_buf, acc.at[loc_list.at[b]],
                                    add=True)

            plsc.subcore_barrier()
            pltpu.sync_copy(
                acc.at[pl.ds(sub * rows_per_tile, rows_per_tile)],
                out_hbm.at[pl.ds(base + sub * rows_per_tile, rows_per_tile)])
            plsc.subcore_barrier()

    return scatter_kernel(mem, val, idx2d)


def kernel(mem, val, idx):
    n_pts = val.shape[0]
    groups = -(-n_pts // (NSUB * LANES))
    p_per_tile = groups * LANES
    pad = NSUB * p_per_tile - n_pts
    idx_pad = jnp.concatenate(
        [idx.astype(jnp.int32), jnp.full((pad,), 1 << 28, jnp.int32)])
    idx2d = idx_pad.reshape(NSUB, p_per_tile)
    return _scatter_add(mem, val, idx2d, p_per_tile)


# speedup vs baseline: 1.3196x; 1.3196x over previous
import dataclasses
import functools

import jax
import jax.numpy as jnp
from jax import lax
from jax.experimental import pallas as pl
from jax.experimental.pallas import tpu as pltpu
from jax.experimental.pallas import tpu_sc as plsc

NCORES = 2
NSUB = 16
LANES = 16

CHUNK = 16384
BLK = 128
BLK_SHIFT = 7
SECS = 2


@functools.partial(jax.jit, static_argnames=("p_per_tile",))
def _scatter_add(mem, val, idx2d, p_per_tile):
    s_rows, feat = mem.shape
    nchunks = s_rows // CHUNK
    ch_per_core = nchunks // NCORES
    p = p_per_tile
    sec_p = p // SECS
    sec_nblk = (sec_p + BLK - 1) // BLK
    rows_per_tile = CHUNK // NSUB
    trash = CHUNK

    mesh = plsc.VectorSubcoreMesh(
        core_axis_name="c", subcore_axis_name="s",
        num_cores=NCORES, num_subcores=NSUB)

    cp = pltpu.CompilerParams(use_tc_tiling_on_sc=False)
    if "needs_layout_passes" in pltpu.CompilerParams.__dataclass_fields__:
        cp = dataclasses.replace(cp, needs_layout_passes=False)

    def tile_body(mem_hbm, val_hbm, idx_hbm, out_hbm, acc,
                  sec_idx, pid_list, loc_list, rows_buf):
        core = lax.axis_index("c")
        sub = lax.axis_index("s")

        ones = jnp.ones((LANES,), jnp.int32)
        zeros = jnp.zeros((LANES,), jnp.int32)
        trash_v = jnp.full((LANES,), trash, jnp.int32)
        lane = lax.iota(jnp.int32, LANES)

        @pl.loop(0, ch_per_core)
        def _chunk(c):
            base = (core * ch_per_core + c) * CHUNK

            pltpu.sync_copy(
                mem_hbm.at[pl.ds(base + sub * rows_per_tile, rows_per_tile)],
                acc.at[pl.ds(sub * rows_per_tile, rows_per_tile)])
            plsc.subcore_barrier()

            @pl.loop(0, SECS)
            def _section(sec):
                sec_base = sec * sec_p
                pltpu.sync_copy(idx_hbm.at[sub].at[pl.ds(sec_base, sec_p)],
                                sec_idx)

                def scan_body(g, n):
                    v = sec_idx[pl.ds(g * LANES, LANES)]
                    loc = v - base
                    mask = (loc >= 0) & (loc < CHUNK)
                    mi = jnp.where(mask, ones, zeros)
                    pos = n + plsc.cumsum(mi) - 1
                    row = lax.shift_right_logical(pos, BLK_SHIFT)
                    col = lax.bitwise_and(pos, BLK - 1)
                    pid = sub * p + sec_base + g * LANES + lane
                    plsc.store_scatter(pid_list, [row, col], pid, mask=mask)
                    plsc.store_scatter(loc_list, [row, col], loc, mask=mask)
                    return n + jnp.sum(mi)

                n = lax.fori_loop(0, sec_p // LANES, scan_body, jnp.int32(0),
                                  unroll=4)

                nb = lax.shift_right_logical(n + (BLK - 1), BLK_SHIFT)
                lim = nb * BLK
                for j in range(BLK // LANES):
                    pos = n + j * LANES + lane
                    mask = pos < lim
                    row = lax.shift_right_logical(pos, BLK_SHIFT)
                    col = lax.bitwise_and(pos, BLK - 1)
                    plsc.store_scatter(pid_list, [row, col], zeros, mask=mask)
                    plsc.store_scatter(loc_list, [row, col], trash_v,
                                       mask=mask)

                @pl.loop(0, sec_nblk)
                def _blk(b):
                    @pl.when(b < nb)
                    def _():
                        pltpu.sync_copy(val_hbm.at[pid_list.at[b]], rows_buf)
                        pltpu.sync_copy(rows_buf, acc.at[loc_list.at[b]],
                                        add=True)

            plsc.subcore_barrier()
            pltpu.sync_copy(
                acc.at[pl.ds(sub * rows_per_tile, rows_per_tile)],
                out_hbm.at[pl.ds(base + sub * rows_per_tile, rows_per_tile)])
            plsc.subcore_barrier()

    @pl.kernel(
        compiler_params=cp,
        out_type=jax.ShapeDtypeStruct((s_rows, feat), jnp.float32),
        mesh=mesh,
        scratch_types=[
            pltpu.VMEM_SHARED((CHUNK + 8, feat), jnp.float32),
        ],
    )
    def scatter_kernel(mem_hbm, val_hbm, idx_hbm, out_hbm, acc):
        pl.run_scoped(
            functools.partial(
                tile_body, mem_hbm, val_hbm, idx_hbm, out_hbm, acc),
            pltpu.VMEM((sec_p,), jnp.int32),
            pltpu.VMEM((sec_nblk, BLK), jnp.int32),
            pltpu.VMEM((sec_nblk, BLK), jnp.int32),
            pltpu.VMEM((BLK, feat), jnp.float32),
        )

    return scatter_kernel(mem, val, idx2d)


def kernel(mem, val, idx):
    n_pts = val.shape[0]
    groups = -(-n_pts // (NSUB * LANES * SECS))
    p_per_tile = groups * LANES * SECS
    pad = NSUB * p_per_tile - n_pts
    idx_pad = jnp.concatenate(
        [idx.astype(jnp.int32), jnp.full((pad,), 1 << 28, jnp.int32)])
    idx2d = idx_pad.reshape(NSUB, p_per_tile)
    return _scatter_add(mem, val, idx2d, p_per_tile)
